# baseline (speedup 1.0000x reference)
import functools

import jax
import jax.numpy as jnp
from jax import lax
from jax.experimental import pallas as pl
from jax.experimental.pallas import tpu as pltpu
from jax.experimental.pallas import tpu_sc as plsc

F32 = jnp.float32
I32 = jnp.int32

N = 10000
E = 320000
NE = 16
NOUT1 = 32
NOUT2 = 16

NC = 2
NS = 16
NW = NC * NS
NPAD = 10240
NQ = 4
NR = NW // NQ
EPAD = 327680
EPR = EPAD // NR
CM = 128
NCHUNK = EPR // CM
AW = NPAD * 8

HI = jax.lax.Precision.HIGHEST


R = 400
GRID = N // R


def _dot(a, b):
    return jax.lax.dot(a, b, precision=HI, preferred_element_type=F32)


def _tc_proj1(x_ref, wc_ref, wa_ref, ba_ref, wb_ref, bb_ref, p_ref, g_ref):
    xb = x_ref[...]
    p_ref[...] = _dot(xb, wc_ref[...])
    a = _dot(xb, wa_ref[...]) + ba_ref[...]
    b = _dot(xb, wb_ref[...]) + bb_ref[...]
    g_ref[...] = jnp.tanh(a) * jnp.tanh(b)


def _ht_blocks(acc_ref, bc_ref):
    hts = []
    for q in range(NQ):
        s = acc_ref[q, 0]
        for r in range(1, NR):
            s = s + acc_ref[q, r]
        hts.append(jnp.maximum(s + bc_ref[0, q], 0.0))
    return hts


def _tc_proj2(acc_ref, g_ref, bc_ref, wct_ref, wcb_ref, wat_ref, wab_ref,
              ba_ref, wbt_ref, wbb_ref, bb_ref, p_ref, gout_ref):
    hts = _ht_blocks(acc_ref, bc_ref)
    hb = g_ref[...]
    p = _dot(hb, wcb_ref[...])
    a = _dot(hb, wab_ref[...]) + ba_ref[...]
    b = _dot(hb, wbb_ref[...]) + bb_ref[...]
    for q in range(NQ):
        p = p + _dot(hts[q], wct_ref[q])
        a = a + _dot(hts[q], wat_ref[q])
        b = b + _dot(hts[q], wbt_ref[q])
    p_ref[...] = p
    gout_ref[...] = jnp.tanh(a) * jnp.tanh(b)


def _tc_final(acc_ref, g_ref, bc_ref, wft_ref, wfb_ref, bf_ref, o_ref):
    hts = _ht_blocks(acc_ref, bc_ref)
    o = jnp.sum(g_ref[...] * wfb_ref[...], axis=1, keepdims=True) + bf_ref[...]
    for q in range(NQ):
        o = o + jnp.sum(hts[q] * wft_ref[0, q][None, :], axis=1, keepdims=True)
    o_ref[...] = o


def _full(idx_map, shape):
    return pl.BlockSpec(shape, idx_map)


_ROW = lambda i: (i, 0)
_W = lambda i: (0, 0)
_W3 = lambda i: (0, 0, 0)
_ACC = lambda i: (0, 0, i, 0)


def _proj1(x, wc, wa, ba, wb, bb, fin):
    return pl.pallas_call(
        _tc_proj1,
        grid=(GRID,),
        in_specs=[
            _full(_ROW, (R, fin)),
            _full(_W, (fin, NE * NOUT1)),
            _full(_W, (fin, NOUT2)),
            _full(_W, (1, NOUT2)),
            _full(_W, (fin, NOUT2)),
            _full(_W, (1, NOUT2)),
        ],
        out_specs=[_full(_ROW, (R, NE * NOUT1)), _full(_ROW, (R, NOUT2))],
        out_shape=[
            jax.ShapeDtypeStruct((N, NE * NOUT1), F32),
            jax.ShapeDtypeStruct((N, NOUT2), F32),
        ],
    )(x, wc, wa, ba, wb, bb)


def _proj2(acc4, g, bc, wct, wat, wbt, wcb, wab, ba, wbb, bb):
    return pl.pallas_call(
        _tc_proj2,
        grid=(GRID,),
        in_specs=[
            pl.BlockSpec((NQ, NR, R, 8), _ACC),
            _full(_ROW, (R, NOUT2)),
            pl.BlockSpec((1, NQ, 8), lambda i: (0, 0, 0)),
            _full(_W3, (NQ, 8, NE * NOUT1)),
            _full(_W, (NOUT2, NE * NOUT1)),
            _full(_W3, (NQ, 8, NOUT2)),
            _full(_W, (NOUT2, NOUT2)),
            _full(_W, (1, NOUT2)),
            _full(_W3, (NQ, 8, NOUT2)),
            _full(_W, (NOUT2, NOUT2)),
            _full(_W, (1, NOUT2)),
        ],
        out_specs=[_full(_ROW, (R, NE * NOUT1)), _full(_ROW, (R, NOUT2))],
        out_shape=[
            jax.ShapeDtypeStruct((N, NE * NOUT1), F32),
            jax.ShapeDtypeStruct((N, NOUT2), F32),
        ],
    )(acc4, g, bc, wct, wcb, wat, wab, ba, wbt, wbb, bb)


def _final(acc4, g, bc, wft, wfb, bf):
    return pl.pallas_call(
        _tc_final,
        grid=(GRID,),
        in_specs=[
            pl.BlockSpec((NQ, NR, R, 8), _ACC),
            _full(_ROW, (R, NOUT2)),
            pl.BlockSpec((1, NQ, 8), lambda i: (0, 0, 0)),
            pl.BlockSpec((1, NQ, 8), lambda i: (0, 0, 0)),
            _full(_W, (1, NOUT2)),
            _full(_W, (1, 1)),
        ],
        out_specs=[_full(_ROW, (R, 1))],
        out_shape=[jax.ShapeDtypeStruct((N, 1), F32)],
    )(acc4, g, bc, wft, wfb, bf)[0]


def _sc_edge_body(p_hbm, src_hbm, dst_hbm, ea_hbm, out_hbm,
                  accf, sbuf0, sbuf1, dbuf0, dbuf1, sidx0, sidx1,
                  ea_v0, ea_v1, rows_v0, rows_v1, msg_v,
                  g_sem0, g_sem1, i_sem0, i_sem1):
    cid = lax.axis_index("c")
    sid = lax.axis_index("s")
    g = cid * NS + sid
    q = g // NR
    r = lax.rem(g, NR)

    sbufs = (sbuf0, sbuf1)
    dbufs = (dbuf0, dbuf1)
    sidxs = (sidx0, sidx1)
    ea_vs = (ea_v0, ea_v1)
    rows_vs = (rows_v0, rows_v1)
    g_sems = (g_sem0, g_sem1)
    i_sems = (i_sem0, i_sem1)

    zero16 = jnp.zeros((16,), F32)
    iota = lax.iota(I32, 16)

    def zero_body(k, carry):
        accf[pl.ds(k * 16, 16)] = zero16
        return carry

    lax.fori_loop(0, (AW + 16) // 16, zero_body, 0)

    def issue_idx(cc, b):
        base = r * EPR + cc * CM
        pltpu.async_copy(src_hbm.at[pl.ds(base, CM)], sbufs[b], i_sems[b])
        pltpu.async_copy(dst_hbm.at[pl.ds(base, CM)],
                         dbufs[b].at[pl.ds(0, CM)], i_sems[b])
        pltpu.async_copy(ea_hbm.at[pl.ds(base * NE, CM * NE)], ea_vs[b],
                         i_sems[b])

    def wait_idx(cc, b):
        base = r * EPR + cc * CM
        pltpu.make_async_copy(src_hbm.at[pl.ds(base, CM)], sbufs[b],
                              i_sems[b]).wait()
        pltpu.make_async_copy(dst_hbm.at[pl.ds(base, CM)],
                              dbufs[b].at[pl.ds(0, CM)], i_sems[b]).wait()
        pltpu.make_async_copy(ea_hbm.at[pl.ds(base * NE, CM * NE)], ea_vs[b],
                              i_sems[b]).wait()

    def issue_gather(b):
        for t in range(CM // 16):
            sidxs[b][pl.ds(16 * t, 16)] = sbufs[b][pl.ds(16 * t, 16)] * NQ + q
        pltpu.async_copy(p_hbm.at[sidxs[b]], rows_vs[b], g_sems[b])

    def wait_gather(b):
        pltpu.make_async_copy(p_hbm.at[sidxs[b]], rows_vs[b],
                              g_sems[b]).wait()

    cvecs = [2 * k + jnp.where(iota < 8, 0, 1) for k in range(8)]
    lane_lo = iota < 8

    def compute(b):
        ea_v, rows_v, dbuf = ea_vs[b], rows_vs[b], dbufs[b]

        @plsc.parallel_loop(0, CM, unroll=8)
        def _(e):
            e16 = jnp.full((16,), e * NE, I32)
            tot = zero16
            tot2 = zero16
            for k in range(0, 8, 2):
                tot = tot + (plsc.load_gather(ea_v, [e16 + cvecs[k]])
                             * rows_v[e, pl.ds(16 * k, 16)])
                tot2 = tot2 + (plsc.load_gather(ea_v, [e16 + cvecs[k + 1]])
                               * rows_v[e, pl.ds(16 * (k + 1), 16)])
            tot = tot + tot2
            m = tot + lax.rev(tot, (0,))
            msg_v[pl.ds(e * 16, 16)] = jnp.where(lane_lo, m, 0.0)

        def rmw_body(t, carry):
            dv = dbuf[pl.ds(16 * t, 16)]
            for j in range(16):
                dstv = dv[j]
                plsc.addupdate(accf.at[pl.ds(dstv * 8, 16)],
                               msg_v[pl.ds((16 * t + j) * 16, 16)])
            return carry

        lax.fori_loop(0, CM // 16, rmw_body, 0)

    issue_idx(0, 0)
    issue_idx(1, 1)
    wait_idx(0, 0)
    issue_gather(0)

    def chunk_pair(it, carry):
        for b in (0, 1):
            cc = 2 * it + b
            wait_idx(cc + 1, 1 - b)
            issue_gather(1 - b)
            wait_gather(b)
            compute(b)
            issue_idx(cc + 2, b)
        return carry

    lax.fori_loop(0, NCHUNK // 2 - 1, chunk_pair, 0)

    wait_idx(NCHUNK - 1, 1)
    issue_gather(1)
    wait_gather(0)
    compute(0)
    wait_gather(1)
    compute(1)

    pltpu.sync_copy(accf.at[pl.ds(0, AW)], out_hbm.at[pl.ds(g * AW, AW)])


def _sc_edge(p2, src1, dst1, ea1):
    mesh = plsc.VectorSubcoreMesh(core_axis_name="c", subcore_axis_name="s")
    f = functools.partial(
        pl.kernel,
        out_type=jax.ShapeDtypeStruct((NW * AW,), F32),
        mesh=mesh,
        scratch_types=[
            pltpu.VMEM((AW + 16,), F32),
            pltpu.VMEM((CM,), I32),
            pltpu.VMEM((CM,), I32),
            pltpu.VMEM((CM + 16,), I32),
            pltpu.VMEM((CM + 16,), I32),
            pltpu.VMEM((CM,), I32),
            pltpu.VMEM((CM,), I32),
            pltpu.VMEM((CM * NE,), F32),
            pltpu.VMEM((CM * NE,), F32),
            pltpu.VMEM((CM, 128), F32),
            pltpu.VMEM((CM, 128), F32),
            pltpu.VMEM((CM * 16,), F32),
            pltpu.SemaphoreType.DMA,
            pltpu.SemaphoreType.DMA,
            pltpu.SemaphoreType.DMA,
            pltpu.SemaphoreType.DMA,
        ],
        compiler_params=pltpu.CompilerParams(needs_layout_passes=False),
    )(_sc_edge_body)
    return f(p2, src1, dst1, ea1)


def _wc_cat(Wc):
    fin = Wc.shape[1]
    t = jnp.transpose(Wc, (1, 0, 2)).reshape(fin, NE, NQ, 8)
    t = t.at[:, 1::2].set(t[:, 1::2, :, ::-1])
    return jnp.transpose(t, (0, 2, 1, 3)).reshape(fin, NE * NOUT1)


def _split_rows(w):
    return w.reshape(NQ, 8, w.shape[1])


def kernel(x, edge_index, edge_attr,
           Wc1, bc1, Wa1, ba1, Wb1, bb1,
           Wc2, bc2, Wa2, ba2, Wb2, bb2,
           Wc3, bc3, Wa3, ba3, Wb3, bb3,
           Wf, bf):
    pad = EPAD - E
    src1 = jnp.pad(edge_index[0], (0, pad))
    dst1 = jnp.pad(edge_index[1], (0, pad))
    ea1 = jnp.pad(edge_attr, ((0, pad), (0, 0))).reshape(-1)

    r2 = lambda v: v.reshape(1, -1)
    racc = lambda a: a.reshape(NQ, NR, NPAD, 8)
    rbc = lambda b: b.reshape(1, NQ, 8)

    p1, g1 = _proj1(x, _wc_cat(Wc1), Wa1, r2(ba1), Wb1, r2(bb1), x.shape[1])
    acc1 = _sc_edge(p1.reshape(NQ * N, 128), src1, dst1, ea1)

    wc2 = _wc_cat(Wc2)
    p2, g2 = _proj2(racc(acc1), g1, rbc(bc1),
                    _split_rows(wc2[:NOUT1]), _split_rows(Wa2[:NOUT1]),
                    _split_rows(Wb2[:NOUT1]),
                    wc2[NOUT1:], Wa2[NOUT1:], r2(ba2), Wb2[NOUT1:], r2(bb2))
    acc2 = _sc_edge(p2.reshape(NQ * N, 128), src1, dst1, ea1)

    wc3 = _wc_cat(Wc3)
    p3, g3 = _proj2(racc(acc2), g2, rbc(bc2),
                    _split_rows(wc3[:NOUT1]), _split_rows(Wa3[:NOUT1]),
                    _split_rows(Wb3[:NOUT1]),
                    wc3[NOUT1:], Wa3[NOUT1:], r2(ba3), Wb3[NOUT1:], r2(bb3))
    acc3 = _sc_edge(p3.reshape(NQ * N, 128), src1, dst1, ea1)

    return _final(racc(acc3), g3, rbc(bc3),
                  Wf[:NOUT1].reshape(1, NQ, 8), Wf[NOUT1:].reshape(1, NOUT2),
                  bf.reshape(1, 1))


# speedup vs baseline: 1.1806x; 1.1806x over previous
import functools

import jax
import jax.numpy as jnp
from jax import lax
from jax.experimental import pallas as pl
from jax.experimental.pallas import tpu as pltpu
from jax.experimental.pallas import tpu_sc as plsc

F32 = jnp.float32
I32 = jnp.int32

N = 10000
E = 320000
NE = 16
NOUT1 = 32
NOUT2 = 16

NC = 2
NS = 16
NW = NC * NS
NPAD = 10240
NQ = 4
NR = NW // NQ
EPAD = 327680
EPR = EPAD // NR
CM = 128
NCHUNK = EPR // CM
AW = NPAD * 8

HI = jax.lax.Precision.HIGHEST


R = 400
GRID = N // R


def _dot(a, b):
    return jax.lax.dot(a, b, precision=HI, preferred_element_type=F32)


def _tc_proj1(x_ref, wc_ref, wa_ref, ba_ref, wb_ref, bb_ref, p_ref, g_ref):
    xb = x_ref[...]
    p_ref[...] = _dot(xb, wc_ref[...])
    a = _dot(xb, wa_ref[...]) + ba_ref[...]
    b = _dot(xb, wb_ref[...]) + bb_ref[...]
    g_ref[...] = jnp.tanh(a) * jnp.tanh(b)


def _ht_blocks(acc_ref, bc_ref):
    hts = []
    for q in range(NQ):
        s = acc_ref[q, 0]
        for r in range(1, NR):
            s = s + acc_ref[q, r]
        hts.append(jnp.maximum(s + bc_ref[0, q], 0.0))
    return hts


def _tc_proj2(acc_ref, g_ref, bc_ref, wct_ref, wcb_ref, wat_ref, wab_ref,
              ba_ref, wbt_ref, wbb_ref, bb_ref, p_ref, gout_ref):
    hts = _ht_blocks(acc_ref, bc_ref)
    hb = g_ref[...]
    p = _dot(hb, wcb_ref[...])
    a = _dot(hb, wab_ref[...]) + ba_ref[...]
    b = _dot(hb, wbb_ref[...]) + bb_ref[...]
    for q in range(NQ):
        p = p + _dot(hts[q], wct_ref[q])
        a = a + _dot(hts[q], wat_ref[q])
        b = b + _dot(hts[q], wbt_ref[q])
    p_ref[...] = p
    gout_ref[...] = jnp.tanh(a) * jnp.tanh(b)


def _tc_final(acc_ref, g_ref, bc_ref, wft_ref, wfb_ref, bf_ref, o_ref):
    hts = _ht_blocks(acc_ref, bc_ref)
    o = jnp.sum(g_ref[...] * wfb_ref[...], axis=1, keepdims=True) + bf_ref[...]
    for q in range(NQ):
        o = o + jnp.sum(hts[q] * wft_ref[0, q][None, :], axis=1, keepdims=True)
    o_ref[...] = o


def _full(idx_map, shape):
    return pl.BlockSpec(shape, idx_map)


_ROW = lambda i: (i, 0)
_W = lambda i: (0, 0)
_W3 = lambda i: (0, 0, 0)
_ACC = lambda i: (0, 0, i, 0)


def _proj1(x, wc, wa, ba, wb, bb, fin):
    return pl.pallas_call(
        _tc_proj1,
        grid=(GRID,),
        in_specs=[
            _full(_ROW, (R, fin)),
            _full(_W, (fin, NE * NOUT1)),
            _full(_W, (fin, NOUT2)),
            _full(_W, (1, NOUT2)),
            _full(_W, (fin, NOUT2)),
            _full(_W, (1, NOUT2)),
        ],
        out_specs=[_full(_ROW, (R, NE * NOUT1)), _full(_ROW, (R, NOUT2))],
        out_shape=[
            jax.ShapeDtypeStruct((N, NE * NOUT1), F32),
            jax.ShapeDtypeStruct((N, NOUT2), F32),
        ],
    )(x, wc, wa, ba, wb, bb)


def _proj2(acc4, g, bc, wct, wat, wbt, wcb, wab, ba, wbb, bb):
    return pl.pallas_call(
        _tc_proj2,
        grid=(GRID,),
        in_specs=[
            pl.BlockSpec((NQ, NR, R, 8), _ACC),
            _full(_ROW, (R, NOUT2)),
            pl.BlockSpec((1, NQ, 8), lambda i: (0, 0, 0)),
            _full(_W3, (NQ, 8, NE * NOUT1)),
            _full(_W, (NOUT2, NE * NOUT1)),
            _full(_W3, (NQ, 8, NOUT2)),
            _full(_W, (NOUT2, NOUT2)),
            _full(_W, (1, NOUT2)),
            _full(_W3, (NQ, 8, NOUT2)),
            _full(_W, (NOUT2, NOUT2)),
            _full(_W, (1, NOUT2)),
        ],
        out_specs=[_full(_ROW, (R, NE * NOUT1)), _full(_ROW, (R, NOUT2))],
        out_shape=[
            jax.ShapeDtypeStruct((N, NE * NOUT1), F32),
            jax.ShapeDtypeStruct((N, NOUT2), F32),
        ],
    )(acc4, g, bc, wct, wcb, wat, wab, ba, wbt, wbb, bb)


def _final(acc4, g, bc, wft, wfb, bf):
    return pl.pallas_call(
        _tc_final,
        grid=(GRID,),
        in_specs=[
            pl.BlockSpec((NQ, NR, R, 8), _ACC),
            _full(_ROW, (R, NOUT2)),
            pl.BlockSpec((1, NQ, 8), lambda i: (0, 0, 0)),
            pl.BlockSpec((1, NQ, 8), lambda i: (0, 0, 0)),
            _full(_W, (1, NOUT2)),
            _full(_W, (1, 1)),
        ],
        out_specs=[_full(_ROW, (R, 1))],
        out_shape=[jax.ShapeDtypeStruct((N, 1), F32)],
    )(acc4, g, bc, wft, wfb, bf)[0]


def _sc_edge_body(p_hbm, src_hbm, dst_hbm, ea_hbm, out_hbm,
                  accf, sbuf0, sbuf1, dbuf0, dbuf1, sidx0, sidx1,
                  ea_v0, ea_v1, rows_v0, rows_v1, msg_v,
                  g_sem0, g_sem1, i_sem0, i_sem1):
    cid = lax.axis_index("c")
    sid = lax.axis_index("s")
    g = cid * NS + sid
    q = g // NR
    r = lax.rem(g, NR)

    sbufs = (sbuf0, sbuf1)
    dbufs = (dbuf0, dbuf1)
    sidxs = (sidx0, sidx1)
    ea_vs = (ea_v0, ea_v1)
    rows_vs = (rows_v0, rows_v1)
    g_sems = (g_sem0, g_sem1)
    i_sems = (i_sem0, i_sem1)

    zero16 = jnp.zeros((16,), F32)
    iota = lax.iota(I32, 16)

    def zero_body(k, carry):
        accf[pl.ds(k * 16, 16)] = zero16
        return carry

    lax.fori_loop(0, (AW + 16) // 16, zero_body, 0)

    def issue_idx(cc, b):
        base = r * EPR + cc * CM
        pltpu.async_copy(src_hbm.at[pl.ds(base, CM)], sbufs[b], i_sems[b])
        pltpu.async_copy(dst_hbm.at[pl.ds(base, CM)],
                         dbufs[b].at[pl.ds(0, CM)], i_sems[b])
        pltpu.async_copy(ea_hbm.at[pl.ds(base * NE, CM * NE)], ea_vs[b],
                         i_sems[b])

    def wait_idx(cc, b):
        base = r * EPR + cc * CM
        pltpu.make_async_copy(src_hbm.at[pl.ds(base, CM)], sbufs[b],
                              i_sems[b]).wait()
        pltpu.make_async_copy(dst_hbm.at[pl.ds(base, CM)],
                              dbufs[b].at[pl.ds(0, CM)], i_sems[b]).wait()
        pltpu.make_async_copy(ea_hbm.at[pl.ds(base * NE, CM * NE)], ea_vs[b],
                              i_sems[b]).wait()

    def issue_gather(b):
        for t in range(CM // 16):
            sidxs[b][pl.ds(16 * t, 16)] = sbufs[b][pl.ds(16 * t, 16)] * NQ + q
        pltpu.async_copy(p_hbm.at[sidxs[b]], rows_vs[b], g_sems[b])

    def wait_gather(b):
        pltpu.make_async_copy(p_hbm.at[sidxs[b]], rows_vs[b],
                              g_sems[b]).wait()

    cvecs = [2 * k + jnp.where(iota < 8, 0, 1) for k in range(8)]
    lane_lo = iota < 8

    def compute(b):
        ea_v, rows_v, dbuf = ea_vs[b], rows_vs[b], dbufs[b]

        @plsc.parallel_loop(0, CM, unroll=4)
        def _(e):
            e16 = jnp.full((16,), e * NE, I32)
            tot = zero16
            tot2 = zero16
            for k in range(0, 8, 2):
                tot = tot + (plsc.load_gather(ea_v, [e16 + cvecs[k]])
                             * rows_v[e, pl.ds(16 * k, 16)])
                tot2 = tot2 + (plsc.load_gather(ea_v, [e16 + cvecs[k + 1]])
                               * rows_v[e, pl.ds(16 * (k + 1), 16)])
            tot = tot + tot2
            m = tot + lax.rev(tot, (0,))
            msg_v[pl.ds(e * 16, 16)] = jnp.where(lane_lo, m, 0.0)

        def rmw_body(t, carry):
            dv = dbuf[pl.ds(16 * t, 16)]
            for j in range(16):
                dstv = dv[j]
                plsc.addupdate(accf.at[pl.ds(dstv * 8, 16)],
                               msg_v[pl.ds((16 * t + j) * 16, 16)])
            return carry

        lax.fori_loop(0, CM // 16, rmw_body, 0)

    issue_idx(0, 0)
    issue_idx(1, 1)
    wait_idx(0, 0)
    issue_gather(0)

    def chunk_pair(it, carry):
        for b in (0, 1):
            cc = 2 * it + b
            wait_idx(cc + 1, 1 - b)
            issue_gather(1 - b)
            wait_gather(b)
            compute(b)
            issue_idx(cc + 2, b)
        return carry

    lax.fori_loop(0, NCHUNK // 2 - 1, chunk_pair, 0)

    wait_idx(NCHUNK - 1, 1)
    issue_gather(1)
    wait_gather(0)
    compute(0)
    wait_gather(1)
    compute(1)

    pltpu.sync_copy(accf.at[pl.ds(0, AW)], out_hbm.at[pl.ds(g * AW, AW)])


def _sc_edge(p2, src1, dst1, ea1):
    mesh = plsc.VectorSubcoreMesh(core_axis_name="c", subcore_axis_name="s")
    f = functools.partial(
        pl.kernel,
        out_type=jax.ShapeDtypeStruct((NW * AW,), F32),
        mesh=mesh,
        scratch_types=[
            pltpu.VMEM((AW + 16,), F32),
            pltpu.VMEM((CM,), I32),
            pltpu.VMEM((CM,), I32),
            pltpu.VMEM((CM + 16,), I32),
            pltpu.VMEM((CM + 16,), I32),
            pltpu.VMEM((CM,), I32),
            pltpu.VMEM((CM,), I32),
            pltpu.VMEM((CM * NE,), F32),
            pltpu.VMEM((CM * NE,), F32),
            pltpu.VMEM((CM, 128), F32),
            pltpu.VMEM((CM, 128), F32),
            pltpu.VMEM((CM * 16,), F32),
            pltpu.SemaphoreType.DMA,
            pltpu.SemaphoreType.DMA,
            pltpu.SemaphoreType.DMA,
            pltpu.SemaphoreType.DMA,
        ],
        compiler_params=pltpu.CompilerParams(needs_layout_passes=False),
    )(_sc_edge_body)
    return f(p2, src1, dst1, ea1)


def _wc_cat(Wc):
    fin = Wc.shape[1]
    t = jnp.transpose(Wc, (1, 0, 2)).reshape(fin, NE, NQ, 8)
    t = t.at[:, 1::2].set(t[:, 1::2, :, ::-1])
    return jnp.transpose(t, (0, 2, 1, 3)).reshape(fin, NE * NOUT1)


def _split_rows(w):
    return w.reshape(NQ, 8, w.shape[1])


def kernel(x, edge_index, edge_attr,
           Wc1, bc1, Wa1, ba1, Wb1, bb1,
           Wc2, bc2, Wa2, ba2, Wb2, bb2,
           Wc3, bc3, Wa3, ba3, Wb3, bb3,
           Wf, bf):
    pad = EPAD - E
    src1 = jnp.pad(edge_index[0], (0, pad))
    dst1 = jnp.pad(edge_index[1], (0, pad))
    ea1 = jnp.pad(edge_attr, ((0, pad), (0, 0))).reshape(-1)

    r2 = lambda v: v.reshape(1, -1)
    racc = lambda a: a.reshape(NQ, NR, NPAD, 8)
    rbc = lambda b: b.reshape(1, NQ, 8)

    p1, g1 = _proj1(x, _wc_cat(Wc1), Wa1, r2(ba1), Wb1, r2(bb1), x.shape[1])
    acc1 = _sc_edge(p1.reshape(NQ * N, 128), src1, dst1, ea1)

    wc2 = _wc_cat(Wc2)
    p2, g2 = _proj2(racc(acc1), g1, rbc(bc1),
                    _split_rows(wc2[:NOUT1]), _split_rows(Wa2[:NOUT1]),
                    _split_rows(Wb2[:NOUT1]),
                    wc2[NOUT1:], Wa2[NOUT1:], r2(ba2), Wb2[NOUT1:], r2(bb2))
    acc2 = _sc_edge(p2.reshape(NQ * N, 128), src1, dst1, ea1)

    wc3 = _wc_cat(Wc3)
    p3, g3 = _proj2(racc(acc2), g2, rbc(bc2),
                    _split_rows(wc3[:NOUT1]), _split_rows(Wa3[:NOUT1]),
                    _split_rows(Wb3[:NOUT1]),
                    wc3[NOUT1:], Wa3[NOUT1:], r2(ba3), Wb3[NOUT1:], r2(bb3))
    acc3 = _sc_edge(p3.reshape(NQ * N, 128), src1, dst1, ea1)

    return _final(racc(acc3), g3, rbc(bc3),
                  Wf[:NOUT1].reshape(1, NQ, 8), Wf[NOUT1:].reshape(1, NOUT2),
                  bf.reshape(1, 1))
